# baseline (speedup 1.0000x reference)
import jax
import jax.numpy as jnp
from jax import lax
from jax.experimental import pallas as pl
from jax.experimental.pallas import tpu as pltpu
from jax.experimental.pallas import tpu_sc as plsc

N = 1000000
C = 32
NUM_OUT = 262144
NUM_RANGES = 8
S = NUM_OUT // NUM_RANGES
PASSES = NUM_RANGES // 2
BLK = 256
NBLK = (N + BLK - 1) // BLK
GPB = BLK // 16
CW = BLK * 3
FW = BLK * C
RING = 512
FB = 128
OB = 64
SPAN = S // 16


def _body(feat_hbm, coords_hbm, out_hbm,
          sums_sh, cnts_sh,
          cbuf, fbuf, srcr_v, brel_v, ring_rel, cr_v,
          idx_rows, ones_st, sums_o, cnts_o, out_stage,
          csems, fsems):
    c = lax.axis_index("c")
    s = lax.axis_index("s")
    iota = lax.iota(jnp.int32, 16)
    f_one = jnp.full((16,), 1.0, jnp.float32)
    f_zero = jnp.full((16,), 0.0, jnp.float32)
    i_zero = jnp.full((16,), 0, jnp.int32)
    i_trash = jnp.full((16,), S, jnp.int32)

    def init_z(i, _):
        ones_st[i] = f_zero
        return 0
    lax.fori_loop(0, FB, init_z, 0)

    n_my = jnp.where(s < NBLK - 16 * (NBLK // 16), NBLK // 16 + 1,
                     NBLK // 16)

    def start_block(k, slot):
        b = s + k * 16
        fo = jnp.minimum(b * FW, N * C - FW)
        pltpu.async_copy(coords_hbm.at[pl.ds(b * CW, CW)],
                         cbuf.at[pl.ds(slot * CW, CW)], csems.at[slot])
        pltpu.async_copy(feat_hbm.at[pl.ds(fo, FW)],
                         fbuf.at[pl.ds(slot * FW, FW)], fsems.at[slot])

    def wait_block(k, slot):
        b = s + k * 16
        fo = jnp.minimum(b * FW, N * C - FW)
        pltpu.make_async_copy(coords_hbm.at[pl.ds(b * CW, CW)],
                              cbuf.at[pl.ds(slot * CW, CW)],
                              csems.at[slot]).wait()
        pltpu.make_async_copy(feat_hbm.at[pl.ds(fo, FW)],
                              fbuf.at[pl.ds(slot * FW, FW)],
                              fsems.at[slot]).wait()

    for p in range(PASSES):
        rng = c * PASSES + p
        base = rng * S

        def zrow(i, _):
            sums_o[i, pl.ds(0, 16)] = f_zero
            sums_o[i, pl.ds(16, 16)] = f_zero
            return 0
        lax.fori_loop(0, OB, zrow, 0)

        def zcrow(i, _):
            cnts_o[i] = f_zero
            return 0
        lax.fori_loop(0, OB // 4, zcrow, 0)

        def zblk(blk, _):
            r0 = pl.multiple_of(s * SPAN + blk * OB, OB)
            pltpu.sync_copy(sums_o, sums_sh.at[pl.ds(r0, OB)])
            pltpu.sync_copy(cnts_o, cnts_sh.at[pl.ds(r0 // 4, OB // 4)])
            return 0
        lax.fori_loop(0, SPAN // OB, zblk, 0)
        plsc.subcore_barrier()

        def fire_one(i, f0):
            h0 = pl.multiple_of((f0 + i * FB) & (RING - 1), FB)
            for t in range(FB // 16):
                rl = ring_rel[pl.ds(h0 + t * 16, 16)]
                idx_rows[0, pl.ds(t * 16, 16)] = rl
                idx_rows[1, pl.ds(t * 16, 16)] = rl >> 2
                plsc.store_scatter(ones_st, [t * 16 + iota, (rl & 3) * 4],
                                   f_one)
            pltpu.sync_copy(cr_v.at[pl.ds(h0, FB)],
                            sums_sh.at[idx_rows.at[0]], add=True)
            pltpu.sync_copy(ones_st, cnts_sh.at[idx_rows.at[1]], add=True)
            for t in range(FB // 16):
                rl = idx_rows[0, pl.ds(t * 16, 16)]
                plsc.store_scatter(ones_st, [t * 16 + iota, (rl & 3) * 4],
                                   f_zero)
            return f0

        def do_block(k, carry):
            off, fired = carry
            slot = k & 1

            @pl.when(k + 1 < n_my)
            def _():
                start_block(k + 1, slot ^ 1)

            wait_block(k, slot)
            c0 = slot * CW
            b = s + k * 16
            shift = b * BLK - jnp.minimum(b * FW, N * C - FW) // C
            f0w = slot * FW + shift * C

            def scan16(g, bs):
                w = c0 + g * 48 + iota * 3
                x = plsc.load_gather(cbuf, [w])
                y = plsc.load_gather(cbuf, [w + 1])
                z = plsc.load_gather(cbuf, [w + 2])
                seg = ((x >> 1) << 12) | ((y >> 1) << 6) | (z >> 1)
                m = (seg >> 15) == rng
                rel = seg & (S - 1)
                plsc.store_compressed(srcr_v.at[pl.ds(bs, 16)], g * 16 + iota,
                                      mask=m)
                plsc.store_compressed(brel_v.at[pl.ds(bs, 16)], rel, mask=m)
                return bs + jnp.sum(m.astype(jnp.int32))

            bs = lax.fori_loop(0, GPB, scan16, 0)

            srcr_v[pl.ds(bs, 16)] = i_zero
            brel_v[pl.ds(bs, 16)] = i_trash

            def compact(gi, _):
                sr = srcr_v[pl.ds(gi * 16, 16)]
                rl = brel_v[pl.ds(gi * 16, 16)]
                pos = (off + gi * 16 + iota) & (RING - 1)
                plsc.store_scatter(ring_rel, [pos], rl)

                def xch(ch, _):
                    v = plsc.load_gather(fbuf, [f0w + sr * C + ch])
                    plsc.store_scatter(cr_v, [pos, i_zero + ch], v)
                    return 0
                lax.fori_loop(0, C, xch, 0)
                return 0

            lax.fori_loop(0, (bs + 15) // 16, compact, 0)
            off = off + bs

            nf = (off - fired) // FB
            lax.fori_loop(0, nf, fire_one, fired)
            return (off, fired + nf * FB)

        start_block(0, 0)
        off, fired = lax.fori_loop(0, n_my, do_block, (0, 0))

        def padrest(j, _):
            pos = (off + j * 16 + iota) & (RING - 1)
            plsc.store_scatter(ring_rel, [pos], i_trash)
            return 0
        lax.fori_loop(0, FB // 16, padrest, 0)
        nf = (off - fired + FB - 1) // FB
        lax.fori_loop(0, nf, fire_one, fired)
        plsc.subcore_barrier()

        def oblk(blk, _):
            r0 = pl.multiple_of(s * SPAN + blk * OB, OB)
            pltpu.sync_copy(sums_sh.at[pl.ds(r0, OB)], sums_o)
            pltpu.sync_copy(cnts_sh.at[pl.ds(r0 // 4, OB // 4)], cnts_o)

            def divrow(rr, _):
                cnt = plsc.load_gather(
                    cnts_o, [i_zero + (rr >> 2), i_zero + (rr & 3) * 4])
                cm = jnp.maximum(cnt, 1.0)
                out_stage[pl.ds(rr * C, 16)] = sums_o[rr, pl.ds(0, 16)] / cm
                out_stage[pl.ds(rr * C + 16, 16)] = \
                    sums_o[rr, pl.ds(16, 16)] / cm
                return 0

            lax.fori_loop(0, OB, divrow, 0)
            pltpu.sync_copy(
                out_stage,
                out_hbm.at[pl.ds(pl.multiple_of((base + r0) * C, 8), OB * C)])
            return 0
        lax.fori_loop(0, SPAN // OB, oblk, 0)
        plsc.subcore_barrier()


@jax.jit
def _pooled(features, coords):
    mesh = plsc.VectorSubcoreMesh(core_axis_name="c", subcore_axis_name="s")
    f = pl.kernel(
        _body,
        out_type=jax.ShapeDtypeStruct((NUM_OUT * C,), jnp.float32),
        mesh=mesh,
        compiler_params=pltpu.CompilerParams(needs_layout_passes=False,
                                             use_tc_tiling_on_sc=False),
        scratch_types=[
            pltpu.VMEM_SHARED((S + 1, C), jnp.float32),
            pltpu.VMEM_SHARED((S // 4 + 1, 16), jnp.float32),
            pltpu.VMEM((2 * CW,), jnp.int32),
            pltpu.VMEM((2 * FW,), jnp.float32),
            pltpu.VMEM((BLK + 16,), jnp.int32),
            pltpu.VMEM((BLK + 16,), jnp.int32),
            pltpu.VMEM((RING,), jnp.int32),
            pltpu.VMEM((RING, C), jnp.float32),
            pltpu.VMEM((2, FB), jnp.int32),
            pltpu.VMEM((FB, 16), jnp.float32),
            pltpu.VMEM((OB, C), jnp.float32),
            pltpu.VMEM((OB // 4, 16), jnp.float32),
            pltpu.VMEM((OB * C,), jnp.float32),
            pltpu.SemaphoreType.DMA((2,)),
            pltpu.SemaphoreType.DMA((2,)),
        ],
    )
    return f(features, coords)


def kernel(features, coords):
    cpad = NBLK * CW - N * 3
    coords1 = jnp.pad(coords.reshape(-1), (0, cpad), constant_values=255)
    one = lax.optimization_barrier(jnp.float32(1.0))
    out = _pooled((features * one).reshape(-1), coords1)
    return out.reshape(NUM_OUT, C)


# speedup vs baseline: 1.0115x; 1.0115x over previous
import jax
import jax.numpy as jnp
from jax import lax
from jax.experimental import pallas as pl
from jax.experimental.pallas import tpu as pltpu
from jax.experimental.pallas import tpu_sc as plsc

N = 1000000
C = 32
NUM_OUT = 262144
NUM_RANGES = 8
S = NUM_OUT // NUM_RANGES
PASSES = NUM_RANGES // 2
BLK = 256
NBLK = (N + BLK - 1) // BLK
GPB = BLK // 16
CW = BLK * 3
FW = BLK * C
RING = 512
FB = 128
OB = 64
SPAN = S // 16


def _body(feat_hbm, coords_hbm, out_hbm,
          sums_sh, cnts_sh,
          cbuf, fbuf, srcr_v, brel_v, ring_rel, cr_v,
          idx_rows, ones_st, sums_o, cnts_o, out_stage,
          csems, fsems):
    c = lax.axis_index("c")
    s = lax.axis_index("s")
    iota = lax.iota(jnp.int32, 16)
    f_one = jnp.full((16,), 1.0, jnp.float32)
    f_zero = jnp.full((16,), 0.0, jnp.float32)
    i_zero = jnp.full((16,), 0, jnp.int32)
    i_trash = jnp.full((16,), S, jnp.int32)

    def init_z(i, _):
        ones_st[i] = f_zero
        return 0
    lax.fori_loop(0, FB, init_z, 0)

    n_my = jnp.where(s < NBLK - 16 * (NBLK // 16), NBLK // 16 + 1,
                     NBLK // 16)

    def start_block(k, slot):
        b = s + k * 16
        fo = jnp.minimum(b * FW, N * C - FW)
        pltpu.async_copy(coords_hbm.at[pl.ds(b * CW, CW)],
                         cbuf.at[pl.ds(slot * CW, CW)], csems.at[slot])
        pltpu.async_copy(feat_hbm.at[pl.ds(fo, FW)],
                         fbuf.at[pl.ds(slot * FW, FW)], fsems.at[slot])

    def wait_block(k, slot):
        b = s + k * 16
        fo = jnp.minimum(b * FW, N * C - FW)
        pltpu.make_async_copy(coords_hbm.at[pl.ds(b * CW, CW)],
                              cbuf.at[pl.ds(slot * CW, CW)],
                              csems.at[slot]).wait()
        pltpu.make_async_copy(feat_hbm.at[pl.ds(fo, FW)],
                              fbuf.at[pl.ds(slot * FW, FW)],
                              fsems.at[slot]).wait()

    for p in range(PASSES):
        rng = c * PASSES + p
        base = rng * S

        def zrow(i, _):
            sums_o[i, pl.ds(0, 16)] = f_zero
            sums_o[i, pl.ds(16, 16)] = f_zero
            return 0
        lax.fori_loop(0, OB, zrow, 0)

        def zcrow(i, _):
            cnts_o[i] = f_zero
            return 0
        lax.fori_loop(0, OB // 4, zcrow, 0)

        def zblk(blk, _):
            r0 = pl.multiple_of(s * SPAN + blk * OB, OB)
            pltpu.sync_copy(sums_o, sums_sh.at[pl.ds(r0, OB)])
            pltpu.sync_copy(cnts_o, cnts_sh.at[pl.ds(r0 // 4, OB // 4)])
            return 0
        lax.fori_loop(0, SPAN // OB, zblk, 0)
        plsc.subcore_barrier()

        def fire_one(i, f0):
            h0 = pl.multiple_of((f0 + i * FB) & (RING - 1), FB)
            for t in range(FB // 16):
                rl = ring_rel[pl.ds(h0 + t * 16, 16)]
                idx_rows[0, pl.ds(t * 16, 16)] = rl
                idx_rows[1, pl.ds(t * 16, 16)] = rl >> 2
                plsc.store_scatter(ones_st, [t * 16 + iota, (rl & 3) * 4],
                                   f_one)
            pltpu.sync_copy(cr_v.at[pl.ds(h0, FB)],
                            sums_sh.at[idx_rows.at[0]], add=True)
            pltpu.sync_copy(ones_st, cnts_sh.at[idx_rows.at[1]], add=True)
            for t in range(FB // 16):
                rl = idx_rows[0, pl.ds(t * 16, 16)]
                plsc.store_scatter(ones_st, [t * 16 + iota, (rl & 3) * 4],
                                   f_zero)
            return f0

        def do_block(k, carry):
            off, fired = carry
            slot = k & 1

            @pl.when(k + 1 < n_my)
            def _():
                start_block(k + 1, slot ^ 1)

            wait_block(k, slot)
            c0 = slot * CW
            b = s + k * 16
            shift = b * BLK - jnp.minimum(b * FW, N * C - FW) // C
            f0w = slot * FW + shift * C

            def scan16(g, bs):
                w = c0 + g * 48 + iota * 3
                x = plsc.load_gather(cbuf, [w])
                y = plsc.load_gather(cbuf, [w + 1])
                z = plsc.load_gather(cbuf, [w + 2])
                seg = ((x >> 1) << 12) | ((y >> 1) << 6) | (z >> 1)
                m = (seg >> 15) == rng
                rel = seg & (S - 1)
                plsc.store_compressed(srcr_v.at[pl.ds(bs, 16)], g * 16 + iota,
                                      mask=m)
                plsc.store_compressed(brel_v.at[pl.ds(bs, 16)], rel, mask=m)
                return bs + jnp.sum(m.astype(jnp.int32))

            bs = lax.fori_loop(0, GPB, scan16, 0)

            srcr_v[pl.ds(bs, 16)] = i_zero
            brel_v[pl.ds(bs, 16)] = i_trash

            def compact(gi, _):
                sr = srcr_v[pl.ds(gi * 16, 16)]
                rl = brel_v[pl.ds(gi * 16, 16)]
                pos = (off + gi * 16 + iota) & (RING - 1)
                plsc.store_scatter(ring_rel, [pos], rl)
                src = f0w + sr * C
                for ch in range(C):
                    v = plsc.load_gather(fbuf, [src + ch])
                    plsc.store_scatter(cr_v, [pos, i_zero + ch], v)
                return 0

            lax.fori_loop(0, (bs + 15) // 16, compact, 0)
            off = off + bs

            nf = (off - fired) // FB
            lax.fori_loop(0, nf, fire_one, fired)
            return (off, fired + nf * FB)

        start_block(0, 0)
        off, fired = lax.fori_loop(0, n_my, do_block, (0, 0))

        def padrest(j, _):
            pos = (off + j * 16 + iota) & (RING - 1)
            plsc.store_scatter(ring_rel, [pos], i_trash)
            return 0
        lax.fori_loop(0, FB // 16, padrest, 0)
        nf = (off - fired + FB - 1) // FB
        lax.fori_loop(0, nf, fire_one, fired)
        plsc.subcore_barrier()

        def oblk(blk, _):
            r0 = pl.multiple_of(s * SPAN + blk * OB, OB)
            pltpu.sync_copy(sums_sh.at[pl.ds(r0, OB)], sums_o)
            pltpu.sync_copy(cnts_sh.at[pl.ds(r0 // 4, OB // 4)], cnts_o)

            def divrow(rr, _):
                cnt = plsc.load_gather(
                    cnts_o, [i_zero + (rr >> 2), i_zero + (rr & 3) * 4])
                cm = jnp.maximum(cnt, 1.0)
                out_stage[pl.ds(rr * C, 16)] = sums_o[rr, pl.ds(0, 16)] / cm
                out_stage[pl.ds(rr * C + 16, 16)] = \
                    sums_o[rr, pl.ds(16, 16)] / cm
                return 0

            lax.fori_loop(0, OB, divrow, 0)
            pltpu.sync_copy(
                out_stage,
                out_hbm.at[pl.ds(pl.multiple_of((base + r0) * C, 8), OB * C)])
            return 0
        lax.fori_loop(0, SPAN // OB, oblk, 0)
        plsc.subcore_barrier()


@jax.jit
def _pooled(features, coords):
    mesh = plsc.VectorSubcoreMesh(core_axis_name="c", subcore_axis_name="s")
    f = pl.kernel(
        _body,
        out_type=jax.ShapeDtypeStruct((NUM_OUT * C,), jnp.float32),
        mesh=mesh,
        compiler_params=pltpu.CompilerParams(needs_layout_passes=False,
                                             use_tc_tiling_on_sc=False),
        scratch_types=[
            pltpu.VMEM_SHARED((S + 1, C), jnp.float32),
            pltpu.VMEM_SHARED((S // 4 + 1, 16), jnp.float32),
            pltpu.VMEM((2 * CW,), jnp.int32),
            pltpu.VMEM((2 * FW,), jnp.float32),
            pltpu.VMEM((BLK + 16,), jnp.int32),
            pltpu.VMEM((BLK + 16,), jnp.int32),
            pltpu.VMEM((RING,), jnp.int32),
            pltpu.VMEM((RING, C), jnp.float32),
            pltpu.VMEM((2, FB), jnp.int32),
            pltpu.VMEM((FB, 16), jnp.float32),
            pltpu.VMEM((OB, C), jnp.float32),
            pltpu.VMEM((OB // 4, 16), jnp.float32),
            pltpu.VMEM((OB * C,), jnp.float32),
            pltpu.SemaphoreType.DMA((2,)),
            pltpu.SemaphoreType.DMA((2,)),
        ],
    )
    return f(features, coords)


def kernel(features, coords):
    cpad = NBLK * CW - N * 3
    coords1 = jnp.pad(coords.reshape(-1), (0, cpad), constant_values=255)
    out = _pooled(features.reshape(-1), coords1)
    return out.reshape(NUM_OUT, C)
